# Initial kernel scaffold; baseline (speedup 1.0000x reference)
import jax
import jax.numpy as jnp
from jax.experimental import pallas as pl


def kernel(x, edge_index, edge_attr, batch, feat_table, W1, b1, g1, be1, W2, b2, eps, edge_tables, g_out, b_out, Wf1, bf1, Wf2, bf2):
    raise NotImplementedError("write your pallas kernel here")


# speedup vs baseline: 12.7370x; 12.7370x over previous
import functools

import jax
import jax.numpy as jnp
from jax import lax
from jax.experimental import pallas as pl
from jax.experimental.pallas import tpu as pltpu
from jax.experimental.pallas import tpu_sc as plsc

N = 10000
EMB = 128
IN_DIM = 128
NUM_EDGE_EMB = 4
NUM_GRAPHS = 128
NUM_TASKS = 10

NC, NS = 2, 16
NW = NC * NS
CH = 64
K = 160
NB = 4
EPW = CH * K
E_PAD = EPW * NW
AGG_ROWS = 10112
ZR = AGG_ROWS // NS
RR = 640


def _sc_scatter_body(idx_hbm, dst_hbm, zeros_hbm, r_hbm, out_hbm,
                     idx_v, rows_v, dst_v, sem_g, sem_s, sem_d, agg_sh):
    c = lax.axis_index("c")
    s = lax.axis_index("s")
    w = c * NS + s

    pltpu.sync_copy(zeros_hbm.at[pl.ds(s * ZR, ZR)], agg_sh.at[pl.ds(s * ZR, ZR)])
    pltpu.sync_copy(idx_hbm.at[w], idx_v)
    plsc.subcore_barrier()

    def fetch(j_row, half, b):
        pltpu.async_copy(
            r_hbm.at[idx_v.at[j_row, pl.ds(half * CH, CH)]], rows_v[b], sem_g[b])
        pltpu.async_copy(dst_hbm.at[w, 2 * j_row + half], dst_v[b], sem_d[b])

    for b in range(NB - 1):
        fetch(b // 2, b % 2, b)

    def body(i, carry):
        for b in range(NB):
            pltpu.make_async_copy(r_hbm.at[pl.ds(0, CH)], rows_v[b], sem_g[b]).wait()
            pltpu.make_async_copy(dst_hbm.at[0, 0], dst_v[b], sem_d[b]).wait()
            pltpu.async_copy(rows_v[b], agg_sh.at[dst_v[b]], sem_s[b], add=True)
            bp = (b - 1) % NB
            jn = NB * i + b + (NB - 1)

            @pl.when(jn < K)
            def _():
                @pl.when(jn >= NB)
                def _():
                    pltpu.make_async_copy(r_hbm.at[pl.ds(0, CH)], rows_v[bp],
                                          sem_s[bp]).wait()

                if b % 2 == 1:
                    fetch((NB * i + b + NB - 1) // 2, 0, bp)
                else:
                    fetch((NB * i + b + NB - 2) // 2, 1, bp)

        return carry

    lax.fori_loop(0, K // NB, body, 0)
    for b in range(NB):
        pltpu.make_async_copy(r_hbm.at[pl.ds(0, CH)], rows_v[b], sem_s[b]).wait()
    plsc.subcore_barrier()

    @pl.when(s < NS - 1)
    def _():
        pltpu.sync_copy(agg_sh.at[pl.ds(s * RR, RR)],
                        out_hbm.at[c, pl.ds(s * RR, RR)])

    @pl.when(s == NS - 1)
    def _():
        pltpu.sync_copy(agg_sh.at[pl.ds((NS - 1) * RR, N - (NS - 1) * RR)],
                        out_hbm.at[c, pl.ds((NS - 1) * RR, N - (NS - 1) * RR)])


@functools.cache
def _get_sc_scatter():
    return pl.kernel(
        _sc_scatter_body,
        out_type=jax.ShapeDtypeStruct((NC, N, EMB), jnp.float32),
        mesh=plsc.VectorSubcoreMesh(core_axis_name="c", subcore_axis_name="s",
                                    num_cores=NC, num_subcores=NS),
        scratch_types=[
            pltpu.VMEM((K // 2, 2 * CH), jnp.int32),
            [pltpu.VMEM((CH, EMB), jnp.float32) for _ in range(NB)],
            [pltpu.VMEM((CH,), jnp.int32) for _ in range(NB)],
            [pltpu.SemaphoreType.DMA for _ in range(NB)],
            [pltpu.SemaphoreType.DMA for _ in range(NB)],
            [pltpu.SemaphoreType.DMA for _ in range(NB)],
            pltpu.VMEM_SHARED((AGG_ROWS, EMB), jnp.float32),
        ],
    )


B = 1000
G = N // B

_full = lambda shape: pl.BlockSpec(shape, lambda i: (0,) * len(shape))
_rows = pl.BlockSpec((B, EMB), lambda i: (i, 0))


def _embed_body(x_ref, ft_ref, tbl_ref, h_ref, r_ref):
    lanes = lax.broadcasted_iota(jnp.int32, (1, IN_DIM), 1)
    onehot = (x_ref[...] == lanes).astype(jnp.float32)
    h = jnp.dot(onehot, ft_ref[...], preferred_element_type=jnp.float32)
    h_ref[...] = h
    r_ref[...] = jnp.maximum(h[None] + tbl_ref[...][:, None, :], 0.0)


_embed = pl.pallas_call(
    _embed_body,
    grid=(G,),
    in_specs=[pl.BlockSpec((B, 1), lambda i: (i, 0)),
              _full((IN_DIM, EMB)), _full((NUM_EDGE_EMB, EMB))],
    out_specs=(_rows, pl.BlockSpec((NUM_EDGE_EMB, B, EMB), lambda i: (0, i, 0))),
    out_shape=(
        jax.ShapeDtypeStruct((N, EMB), jnp.float32),
        jax.ShapeDtypeStruct((NUM_EDGE_EMB, N, EMB), jnp.float32),
    ),
)


def _k1_body(h_ref, p_ref, eps_ref, w1_ref, b1_ref, u_ref, st_ref):
    z = (1.0 + eps_ref[0, 0]) * h_ref[...] + p_ref[0] + p_ref[1]
    u = jnp.dot(z, w1_ref[...], preferred_element_type=jnp.float32) + b1_ref[...]
    u_ref[...] = u

    @pl.when(pl.program_id(0) == 0)
    def _():
        st_ref[...] = jnp.zeros_like(st_ref)

    st_ref[0:1] += jnp.sum(u, axis=0, keepdims=True)
    st_ref[1:2] += jnp.sum(u * u, axis=0, keepdims=True)


_k1 = pl.pallas_call(
    _k1_body,
    grid=(G,),
    in_specs=[_rows, pl.BlockSpec((NC, B, EMB), lambda i: (0, i, 0)),
              pl.BlockSpec(memory_space=pltpu.SMEM),
              _full((EMB, EMB)), _full((1, EMB))],
    out_specs=(_rows, _full((2, EMB))),
    out_shape=(
        jax.ShapeDtypeStruct((N, EMB), jnp.float32),
        jax.ShapeDtypeStruct((2, EMB), jnp.float32),
    ),
)


def _bn_from_stats(u, st, gamma, beta):
    mu = st[0:1] * (1.0 / N)
    var = st[1:2] * (1.0 / N) - mu * mu
    return (u - mu) / jnp.sqrt(var + 1e-5) * gamma + beta


def _k2_body(u_ref, st1_ref, g1_ref, be1_ref, w2_ref, b2_ref, v_ref, st_ref):
    t = jnp.maximum(
        _bn_from_stats(u_ref[...], st1_ref[...], g1_ref[...], be1_ref[...]), 0.0)
    v = jnp.dot(t, w2_ref[...], preferred_element_type=jnp.float32) + b2_ref[...]
    v_ref[...] = v

    @pl.when(pl.program_id(0) == 0)
    def _():
        st_ref[...] = jnp.zeros_like(st_ref)

    st_ref[0:1] += jnp.sum(v, axis=0, keepdims=True)
    st_ref[1:2] += jnp.sum(v * v, axis=0, keepdims=True)


_k2 = pl.pallas_call(
    _k2_body,
    grid=(G,),
    in_specs=[_rows, _full((2, EMB)), _full((1, EMB)), _full((1, EMB)),
              _full((EMB, EMB)), _full((1, EMB))],
    out_specs=(_rows, _full((2, EMB))),
    out_shape=(
        jax.ShapeDtypeStruct((N, EMB), jnp.float32),
        jax.ShapeDtypeStruct((2, EMB), jnp.float32),
    ),
)


def _k3_body(v_ref, st2_ref, go_ref, bo_ref, tbl_ref, hn_ref, r_ref):
    hn = jnp.maximum(
        _bn_from_stats(v_ref[...], st2_ref[...], go_ref[...], bo_ref[...]), 0.0)
    hn_ref[...] = hn
    r_ref[...] = jnp.maximum(hn[None] + tbl_ref[...][:, None, :], 0.0)


_k3 = pl.pallas_call(
    _k3_body,
    grid=(G,),
    in_specs=[_rows, _full((2, EMB)), _full((1, EMB)), _full((1, EMB)),
              _full((NUM_EDGE_EMB, EMB))],
    out_specs=(_rows, pl.BlockSpec((NUM_EDGE_EMB, B, EMB), lambda i: (0, i, 0))),
    out_shape=(
        jax.ShapeDtypeStruct((N, EMB), jnp.float32),
        jax.ShapeDtypeStruct((NUM_EDGE_EMB, N, EMB), jnp.float32),
    ),
)


def _k3f_body(v_ref, st2_ref, go_ref, bo_ref, batch_ref, pool_ref):
    hn = jnp.maximum(
        _bn_from_stats(v_ref[...], st2_ref[...], go_ref[...], bo_ref[...]), 0.0)
    gl = lax.broadcasted_iota(jnp.int32, (1, NUM_GRAPHS), 1)
    onehot = (batch_ref[...] == gl).astype(jnp.float32)

    @pl.when(pl.program_id(0) == 0)
    def _():
        pool_ref[...] = jnp.zeros_like(pool_ref)

    pool_ref[...] += lax.dot_general(onehot, hn, (((0,), (0,)), ((), ())),
                                     preferred_element_type=jnp.float32)


_k3f = pl.pallas_call(
    _k3f_body,
    grid=(G,),
    in_specs=[_rows, _full((2, EMB)), _full((1, EMB)), _full((1, EMB)),
              pl.BlockSpec((B, 1), lambda i: (i, 0))],
    out_specs=_full((NUM_GRAPHS, EMB)),
    out_shape=jax.ShapeDtypeStruct((NUM_GRAPHS, EMB), jnp.float32),
)


def _head_body(pool_ref, wf1_ref, bf1_ref, wf2_ref, bf2_ref, out_ref):
    t = jnp.maximum(
        jnp.dot(pool_ref[...], wf1_ref[...], preferred_element_type=jnp.float32)
        + bf1_ref[...], 0.0)
    out_ref[...] = (jnp.dot(t, wf2_ref[...], preferred_element_type=jnp.float32)
                    + bf2_ref[...])


_head = pl.pallas_call(
    _head_body,
    out_shape=jax.ShapeDtypeStruct((NUM_GRAPHS, NUM_TASKS), jnp.float32),
)


def kernel(x, edge_index, edge_attr, batch, feat_table, W1, b1, g1, be1, W2, b2,
           eps, edge_tables, g_out, b_out, Wf1, bf1, Wf2, bf2):
    e = edge_index.shape[1]
    src = edge_index[0].astype(jnp.int32)
    dst = edge_index[1].astype(jnp.int32)
    idx = edge_attr.astype(jnp.int32) * N + src
    pad = jnp.arange(E_PAD - e, dtype=jnp.int32)
    idx_p = jnp.concatenate(
        [idx, pad % (NUM_EDGE_EMB * N)]).reshape(NW, K // 2, 2 * CH)
    dst_p = jnp.concatenate(
        [dst, N + pad % (AGG_ROWS - N)]).reshape(NW, K, CH)
    zeros = jnp.zeros((AGG_ROWS, EMB), jnp.float32)
    x2 = x.astype(jnp.int32).reshape(N, 1)
    batch2 = batch.astype(jnp.int32).reshape(N, 1)

    h, r = _embed(x2, feat_table, edge_tables[0])
    for l in range(3):
        parts = _get_sc_scatter()(idx_p, dst_p, zeros,
                                  r.reshape(NUM_EDGE_EMB * N, EMB))
        eps_l = eps[l].reshape(1, 1)
        u, st1 = _k1(h, parts, eps_l, W1[l], b1[l].reshape(1, EMB))
        v, st2 = _k2(u, st1, g1[l].reshape(1, EMB), be1[l].reshape(1, EMB),
                     W2[l], b2[l].reshape(1, EMB))
        if l < 2:
            h, r = _k3(v, st2, g_out[l].reshape(1, EMB),
                       b_out[l].reshape(1, EMB), edge_tables[l + 1])
        else:
            pooled = _k3f(v, st2, g_out[l].reshape(1, EMB),
                          b_out[l].reshape(1, EMB), batch2)
    return _head(pooled, Wf1, bf1.reshape(1, 2 * EMB), Wf2,
                 bf2.reshape(1, NUM_TASKS))


# speedup vs baseline: 12.7396x; 1.0002x over previous
import functools

import jax
import jax.numpy as jnp
from jax import lax
from jax.experimental import pallas as pl
from jax.experimental.pallas import tpu as pltpu
from jax.experimental.pallas import tpu_sc as plsc

N = 10000
EMB = 128
IN_DIM = 128
NUM_EDGE_EMB = 4
NUM_GRAPHS = 128
NUM_TASKS = 10

NC, NS = 2, 16
NW = NC * NS
CH = 64
K = 160
NB = 4
CPR = 128 // CH
EPW = CH * K
E_PAD = EPW * NW
AGG_ROWS = 10112
ZR = AGG_ROWS // NS
RR = 640


def _sc_scatter_body(idx_hbm, dst_hbm, zeros_hbm, r_hbm, out_hbm,
                     idx_v, rows_v, dst_v, sem_g, sem_s, sem_d, agg_sh):
    c = lax.axis_index("c")
    s = lax.axis_index("s")
    w = c * NS + s

    pltpu.sync_copy(zeros_hbm.at[pl.ds(s * ZR, ZR)], agg_sh.at[pl.ds(s * ZR, ZR)])
    pltpu.sync_copy(idx_hbm.at[w], idx_v)
    plsc.subcore_barrier()

    def fetch(j_row, part, b):
        pltpu.async_copy(
            r_hbm.at[idx_v.at[j_row, pl.ds(part * CH, CH)]], rows_v[b], sem_g[b])
        pltpu.async_copy(dst_hbm.at[w, CPR * j_row + part], dst_v[b], sem_d[b])

    for b in range(NB - 1):
        fetch(b // CPR, b % CPR, b)

    def body(i, carry):
        for b in range(NB):
            pltpu.make_async_copy(r_hbm.at[pl.ds(0, CH)], rows_v[b], sem_g[b]).wait()
            pltpu.make_async_copy(dst_hbm.at[0, 0], dst_v[b], sem_d[b]).wait()
            pltpu.async_copy(rows_v[b], agg_sh.at[dst_v[b]], sem_s[b], add=True)
            bp = (b - 1) % NB
            jn = NB * i + b + (NB - 1)

            @pl.when(jn < K)
            def _():
                @pl.when(jn >= NB)
                def _():
                    pltpu.make_async_copy(r_hbm.at[pl.ds(0, CH)], rows_v[bp],
                                          sem_s[bp]).wait()

                fetch((NB // CPR) * i + (b + NB - 1) // CPR,
                      (b + NB - 1) % CPR, bp)

        return carry

    lax.fori_loop(0, K // NB, body, 0)
    for b in range(NB):
        pltpu.make_async_copy(r_hbm.at[pl.ds(0, CH)], rows_v[b], sem_s[b]).wait()
    plsc.subcore_barrier()

    @pl.when(s < NS - 1)
    def _():
        pltpu.sync_copy(agg_sh.at[pl.ds(s * RR, RR)],
                        out_hbm.at[c, pl.ds(s * RR, RR)])

    @pl.when(s == NS - 1)
    def _():
        pltpu.sync_copy(agg_sh.at[pl.ds((NS - 1) * RR, N - (NS - 1) * RR)],
                        out_hbm.at[c, pl.ds((NS - 1) * RR, N - (NS - 1) * RR)])


@functools.cache
def _get_sc_scatter():
    return pl.kernel(
        _sc_scatter_body,
        out_type=jax.ShapeDtypeStruct((NC, N, EMB), jnp.float32),
        mesh=plsc.VectorSubcoreMesh(core_axis_name="c", subcore_axis_name="s",
                                    num_cores=NC, num_subcores=NS),
        scratch_types=[
            pltpu.VMEM((K // CPR, CPR * CH), jnp.int32),
            [pltpu.VMEM((CH, EMB), jnp.float32) for _ in range(NB)],
            [pltpu.VMEM((CH,), jnp.int32) for _ in range(NB)],
            [pltpu.SemaphoreType.DMA for _ in range(NB)],
            [pltpu.SemaphoreType.DMA for _ in range(NB)],
            [pltpu.SemaphoreType.DMA for _ in range(NB)],
            pltpu.VMEM_SHARED((AGG_ROWS, EMB), jnp.float32),
        ],
    )


B = 1000
G = N // B

_full = lambda shape: pl.BlockSpec(shape, lambda i: (0,) * len(shape))
_rows = pl.BlockSpec((B, EMB), lambda i: (i, 0))


def _embed_body(x_ref, ft_ref, tbl_ref, h_ref, r_ref):
    lanes = lax.broadcasted_iota(jnp.int32, (1, IN_DIM), 1)
    onehot = (x_ref[...] == lanes).astype(jnp.float32)
    h = jnp.dot(onehot, ft_ref[...], preferred_element_type=jnp.float32)
    h_ref[...] = h
    r_ref[...] = jnp.maximum(h[None] + tbl_ref[...][:, None, :], 0.0)


_embed = pl.pallas_call(
    _embed_body,
    grid=(G,),
    in_specs=[pl.BlockSpec((B, 1), lambda i: (i, 0)),
              _full((IN_DIM, EMB)), _full((NUM_EDGE_EMB, EMB))],
    out_specs=(_rows, pl.BlockSpec((NUM_EDGE_EMB, B, EMB), lambda i: (0, i, 0))),
    out_shape=(
        jax.ShapeDtypeStruct((N, EMB), jnp.float32),
        jax.ShapeDtypeStruct((NUM_EDGE_EMB, N, EMB), jnp.float32),
    ),
)


def _k1_body(h_ref, p_ref, eps_ref, w1_ref, b1_ref, u_ref, st_ref):
    z = (1.0 + eps_ref[0, 0]) * h_ref[...] + p_ref[0] + p_ref[1]
    u = jnp.dot(z, w1_ref[...], preferred_element_type=jnp.float32) + b1_ref[...]
    u_ref[...] = u

    @pl.when(pl.program_id(0) == 0)
    def _():
        st_ref[...] = jnp.zeros_like(st_ref)

    st_ref[0:1] += jnp.sum(u, axis=0, keepdims=True)
    st_ref[1:2] += jnp.sum(u * u, axis=0, keepdims=True)


_k1 = pl.pallas_call(
    _k1_body,
    grid=(G,),
    in_specs=[_rows, pl.BlockSpec((NC, B, EMB), lambda i: (0, i, 0)),
              pl.BlockSpec(memory_space=pltpu.SMEM),
              _full((EMB, EMB)), _full((1, EMB))],
    out_specs=(_rows, _full((2, EMB))),
    out_shape=(
        jax.ShapeDtypeStruct((N, EMB), jnp.float32),
        jax.ShapeDtypeStruct((2, EMB), jnp.float32),
    ),
)


def _bn_from_stats(u, st, gamma, beta):
    mu = st[0:1] * (1.0 / N)
    var = st[1:2] * (1.0 / N) - mu * mu
    return (u - mu) / jnp.sqrt(var + 1e-5) * gamma + beta


def _k2_body(u_ref, st1_ref, g1_ref, be1_ref, w2_ref, b2_ref, v_ref, st_ref):
    t = jnp.maximum(
        _bn_from_stats(u_ref[...], st1_ref[...], g1_ref[...], be1_ref[...]), 0.0)
    v = jnp.dot(t, w2_ref[...], preferred_element_type=jnp.float32) + b2_ref[...]
    v_ref[...] = v

    @pl.when(pl.program_id(0) == 0)
    def _():
        st_ref[...] = jnp.zeros_like(st_ref)

    st_ref[0:1] += jnp.sum(v, axis=0, keepdims=True)
    st_ref[1:2] += jnp.sum(v * v, axis=0, keepdims=True)


_k2 = pl.pallas_call(
    _k2_body,
    grid=(G,),
    in_specs=[_rows, _full((2, EMB)), _full((1, EMB)), _full((1, EMB)),
              _full((EMB, EMB)), _full((1, EMB))],
    out_specs=(_rows, _full((2, EMB))),
    out_shape=(
        jax.ShapeDtypeStruct((N, EMB), jnp.float32),
        jax.ShapeDtypeStruct((2, EMB), jnp.float32),
    ),
)


def _k3_body(v_ref, st2_ref, go_ref, bo_ref, tbl_ref, hn_ref, r_ref):
    hn = jnp.maximum(
        _bn_from_stats(v_ref[...], st2_ref[...], go_ref[...], bo_ref[...]), 0.0)
    hn_ref[...] = hn
    r_ref[...] = jnp.maximum(hn[None] + tbl_ref[...][:, None, :], 0.0)


_k3 = pl.pallas_call(
    _k3_body,
    grid=(G,),
    in_specs=[_rows, _full((2, EMB)), _full((1, EMB)), _full((1, EMB)),
              _full((NUM_EDGE_EMB, EMB))],
    out_specs=(_rows, pl.BlockSpec((NUM_EDGE_EMB, B, EMB), lambda i: (0, i, 0))),
    out_shape=(
        jax.ShapeDtypeStruct((N, EMB), jnp.float32),
        jax.ShapeDtypeStruct((NUM_EDGE_EMB, N, EMB), jnp.float32),
    ),
)


def _k3f_body(v_ref, st2_ref, go_ref, bo_ref, batch_ref, pool_ref):
    hn = jnp.maximum(
        _bn_from_stats(v_ref[...], st2_ref[...], go_ref[...], bo_ref[...]), 0.0)
    gl = lax.broadcasted_iota(jnp.int32, (1, NUM_GRAPHS), 1)
    onehot = (batch_ref[...] == gl).astype(jnp.float32)

    @pl.when(pl.program_id(0) == 0)
    def _():
        pool_ref[...] = jnp.zeros_like(pool_ref)

    pool_ref[...] += lax.dot_general(onehot, hn, (((0,), (0,)), ((), ())),
                                     preferred_element_type=jnp.float32)


_k3f = pl.pallas_call(
    _k3f_body,
    grid=(G,),
    in_specs=[_rows, _full((2, EMB)), _full((1, EMB)), _full((1, EMB)),
              pl.BlockSpec((B, 1), lambda i: (i, 0))],
    out_specs=_full((NUM_GRAPHS, EMB)),
    out_shape=jax.ShapeDtypeStruct((NUM_GRAPHS, EMB), jnp.float32),
)


def _head_body(pool_ref, wf1_ref, bf1_ref, wf2_ref, bf2_ref, out_ref):
    t = jnp.maximum(
        jnp.dot(pool_ref[...], wf1_ref[...], preferred_element_type=jnp.float32)
        + bf1_ref[...], 0.0)
    out_ref[...] = (jnp.dot(t, wf2_ref[...], preferred_element_type=jnp.float32)
                    + bf2_ref[...])


_head = pl.pallas_call(
    _head_body,
    out_shape=jax.ShapeDtypeStruct((NUM_GRAPHS, NUM_TASKS), jnp.float32),
)


def kernel(x, edge_index, edge_attr, batch, feat_table, W1, b1, g1, be1, W2, b2,
           eps, edge_tables, g_out, b_out, Wf1, bf1, Wf2, bf2):
    e = edge_index.shape[1]
    src = edge_index[0].astype(jnp.int32)
    dst = edge_index[1].astype(jnp.int32)
    idx = edge_attr.astype(jnp.int32) * N + src
    pad = jnp.arange(E_PAD - e, dtype=jnp.int32)
    idx_p = jnp.concatenate(
        [idx, pad % (NUM_EDGE_EMB * N)]).reshape(NW, K // CPR, CPR * CH)
    dst_p = jnp.concatenate(
        [dst, N + pad % (AGG_ROWS - N)]).reshape(NW, K, CH)
    zeros = jnp.zeros((AGG_ROWS, EMB), jnp.float32)
    x2 = x.astype(jnp.int32).reshape(N, 1)
    batch2 = batch.astype(jnp.int32).reshape(N, 1)

    h, r = _embed(x2, feat_table, edge_tables[0])
    for l in range(3):
        parts = _get_sc_scatter()(idx_p, dst_p, zeros,
                                  r.reshape(NUM_EDGE_EMB * N, EMB))
        eps_l = eps[l].reshape(1, 1)
        u, st1 = _k1(h, parts, eps_l, W1[l], b1[l].reshape(1, EMB))
        v, st2 = _k2(u, st1, g1[l].reshape(1, EMB), be1[l].reshape(1, EMB),
                     W2[l], b2[l].reshape(1, EMB))
        if l < 2:
            h, r = _k3(v, st2, g_out[l].reshape(1, EMB),
                       b_out[l].reshape(1, EMB), edge_tables[l + 1])
        else:
            pooled = _k3f(v, st2, g_out[l].reshape(1, EMB),
                          b_out[l].reshape(1, EMB), batch2)
    return _head(pooled, Wf1, bf1.reshape(1, 2 * EMB), Wf2,
                 bf2.reshape(1, NUM_TASKS))


# speedup vs baseline: 12.7748x; 1.0028x over previous
import functools

import jax
import jax.numpy as jnp
from jax import lax
from jax.experimental import pallas as pl
from jax.experimental.pallas import tpu as pltpu
from jax.experimental.pallas import tpu_sc as plsc

N = 10000
EMB = 128
IN_DIM = 128
NUM_EDGE_EMB = 4
NUM_GRAPHS = 128
NUM_TASKS = 10

NC, NS = 2, 16
NW = NC * NS
CH = 64
K = 160
NB = 4
CPR = 128 // CH
EPW = CH * K
E_PAD = EPW * NW
AGG_ROWS = 10112
ZR = AGG_ROWS // NS
RR = 640


def _sc_scatter_body(idx_hbm, dst_hbm, zeros_hbm, r_hbm, out_hbm,
                     idx_v, rows_v, dstb, sem_g, sem_s, sem_d, agg_sh):
    c = lax.axis_index("c")
    s = lax.axis_index("s")
    w = c * NS + s

    pltpu.sync_copy(zeros_hbm.at[pl.ds(s * ZR, ZR)], agg_sh.at[pl.ds(s * ZR, ZR)])
    pltpu.sync_copy(idx_hbm.at[w], idx_v)
    plsc.subcore_barrier()

    def fetch(j_row, part, b):
        pltpu.async_copy(
            r_hbm.at[idx_v.at[j_row, pl.ds(part * CH, CH)]], rows_v[b], sem_g[b])

    pltpu.async_copy(dst_hbm.at[w, 0], dstb[0], sem_d[0])
    for b in range(NB - 1):
        fetch(b // CPR, b % CPR, b)

    def cycle(i, parity):
        for b in range(NB):
            pltpu.make_async_copy(r_hbm.at[pl.ds(0, CH)], rows_v[b], sem_g[b]).wait()
            if b == 0:
                pltpu.make_async_copy(dst_hbm.at[0, 0], dstb[parity],
                                      sem_d[parity]).wait()
            pltpu.async_copy(
                rows_v[b],
                agg_sh.at[dstb[parity].at[b // 2, pl.ds((b % 2) * CH, CH)]],
                sem_s[b], add=True)
            bp = (b - 1) % NB
            jn = NB * i + b + (NB - 1)

            @pl.when(jn < K)
            def _():
                @pl.when(jn >= NB)
                def _():
                    pltpu.make_async_copy(r_hbm.at[pl.ds(0, CH)], rows_v[bp],
                                          sem_s[bp]).wait()

                fetch((NB // CPR) * i + (b + NB - 1) // CPR,
                      (b + NB - 1) % CPR, bp)

            if b == 0:
                @pl.when(i + 1 < K // NB)
                def _():
                    pltpu.async_copy(dst_hbm.at[w, i + 1], dstb[1 - parity],
                                     sem_d[1 - parity])

    def body2(t, carry):
        cycle(2 * t, 0)
        cycle(2 * t + 1, 1)
        return carry

    lax.fori_loop(0, K // NB // 2, body2, 0)
    for b in range(NB):
        pltpu.make_async_copy(r_hbm.at[pl.ds(0, CH)], rows_v[b], sem_s[b]).wait()
    plsc.subcore_barrier()

    @pl.when(s < NS - 1)
    def _():
        pltpu.sync_copy(agg_sh.at[pl.ds(s * RR, RR)],
                        out_hbm.at[c, pl.ds(s * RR, RR)])

    @pl.when(s == NS - 1)
    def _():
        pltpu.sync_copy(agg_sh.at[pl.ds((NS - 1) * RR, N - (NS - 1) * RR)],
                        out_hbm.at[c, pl.ds((NS - 1) * RR, N - (NS - 1) * RR)])


@functools.cache
def _get_sc_scatter():
    return pl.kernel(
        _sc_scatter_body,
        out_type=jax.ShapeDtypeStruct((NC, N, EMB), jnp.float32),
        mesh=plsc.VectorSubcoreMesh(core_axis_name="c", subcore_axis_name="s",
                                    num_cores=NC, num_subcores=NS),
        scratch_types=[
            pltpu.VMEM((K // CPR, CPR * CH), jnp.int32),
            [pltpu.VMEM((CH, EMB), jnp.float32) for _ in range(NB)],
            [pltpu.VMEM((2, 2 * CH), jnp.int32) for _ in range(2)],
            [pltpu.SemaphoreType.DMA for _ in range(NB)],
            [pltpu.SemaphoreType.DMA for _ in range(NB)],
            [pltpu.SemaphoreType.DMA for _ in range(2)],
            pltpu.VMEM_SHARED((AGG_ROWS, EMB), jnp.float32),
        ],
    )


B = 1000
G = N // B

_full = lambda shape: pl.BlockSpec(shape, lambda i: (0,) * len(shape))
_rows = pl.BlockSpec((B, EMB), lambda i: (i, 0))


def _embed_body(x_ref, ft_ref, tbl_ref, h_ref, r_ref):
    lanes = lax.broadcasted_iota(jnp.int32, (1, IN_DIM), 1)
    onehot = (x_ref[...] == lanes).astype(jnp.float32)
    h = jnp.dot(onehot, ft_ref[...], preferred_element_type=jnp.float32)
    h_ref[...] = h
    r_ref[...] = jnp.maximum(h[None] + tbl_ref[...][:, None, :], 0.0)


_embed = pl.pallas_call(
    _embed_body,
    grid=(G,),
    in_specs=[pl.BlockSpec((B, 1), lambda i: (i, 0)),
              _full((IN_DIM, EMB)), _full((NUM_EDGE_EMB, EMB))],
    out_specs=(_rows, pl.BlockSpec((NUM_EDGE_EMB, B, EMB), lambda i: (0, i, 0))),
    out_shape=(
        jax.ShapeDtypeStruct((N, EMB), jnp.float32),
        jax.ShapeDtypeStruct((NUM_EDGE_EMB, N, EMB), jnp.float32),
    ),
)


def _k1_body(h_ref, p_ref, eps_ref, w1_ref, b1_ref, u_ref, st_ref):
    z = (1.0 + eps_ref[0, 0]) * h_ref[...] + p_ref[0] + p_ref[1]
    u = jnp.dot(z, w1_ref[...], preferred_element_type=jnp.float32) + b1_ref[...]
    u_ref[...] = u

    @pl.when(pl.program_id(0) == 0)
    def _():
        st_ref[...] = jnp.zeros_like(st_ref)

    st_ref[0:1] += jnp.sum(u, axis=0, keepdims=True)
    st_ref[1:2] += jnp.sum(u * u, axis=0, keepdims=True)


_k1 = pl.pallas_call(
    _k1_body,
    grid=(G,),
    in_specs=[_rows, pl.BlockSpec((NC, B, EMB), lambda i: (0, i, 0)),
              pl.BlockSpec(memory_space=pltpu.SMEM),
              _full((EMB, EMB)), _full((1, EMB))],
    out_specs=(_rows, _full((2, EMB))),
    out_shape=(
        jax.ShapeDtypeStruct((N, EMB), jnp.float32),
        jax.ShapeDtypeStruct((2, EMB), jnp.float32),
    ),
)


def _bn_from_stats(u, st, gamma, beta):
    mu = st[0:1] * (1.0 / N)
    var = st[1:2] * (1.0 / N) - mu * mu
    return (u - mu) / jnp.sqrt(var + 1e-5) * gamma + beta


def _k2_body(u_ref, st1_ref, g1_ref, be1_ref, w2_ref, b2_ref, v_ref, st_ref):
    t = jnp.maximum(
        _bn_from_stats(u_ref[...], st1_ref[...], g1_ref[...], be1_ref[...]), 0.0)
    v = jnp.dot(t, w2_ref[...], preferred_element_type=jnp.float32) + b2_ref[...]
    v_ref[...] = v

    @pl.when(pl.program_id(0) == 0)
    def _():
        st_ref[...] = jnp.zeros_like(st_ref)

    st_ref[0:1] += jnp.sum(v, axis=0, keepdims=True)
    st_ref[1:2] += jnp.sum(v * v, axis=0, keepdims=True)


_k2 = pl.pallas_call(
    _k2_body,
    grid=(G,),
    in_specs=[_rows, _full((2, EMB)), _full((1, EMB)), _full((1, EMB)),
              _full((EMB, EMB)), _full((1, EMB))],
    out_specs=(_rows, _full((2, EMB))),
    out_shape=(
        jax.ShapeDtypeStruct((N, EMB), jnp.float32),
        jax.ShapeDtypeStruct((2, EMB), jnp.float32),
    ),
)


def _k3_body(v_ref, st2_ref, go_ref, bo_ref, tbl_ref, hn_ref, r_ref):
    hn = jnp.maximum(
        _bn_from_stats(v_ref[...], st2_ref[...], go_ref[...], bo_ref[...]), 0.0)
    hn_ref[...] = hn
    r_ref[...] = jnp.maximum(hn[None] + tbl_ref[...][:, None, :], 0.0)


_k3 = pl.pallas_call(
    _k3_body,
    grid=(G,),
    in_specs=[_rows, _full((2, EMB)), _full((1, EMB)), _full((1, EMB)),
              _full((NUM_EDGE_EMB, EMB))],
    out_specs=(_rows, pl.BlockSpec((NUM_EDGE_EMB, B, EMB), lambda i: (0, i, 0))),
    out_shape=(
        jax.ShapeDtypeStruct((N, EMB), jnp.float32),
        jax.ShapeDtypeStruct((NUM_EDGE_EMB, N, EMB), jnp.float32),
    ),
)


def _k3f_body(v_ref, st2_ref, go_ref, bo_ref, batch_ref, pool_ref):
    hn = jnp.maximum(
        _bn_from_stats(v_ref[...], st2_ref[...], go_ref[...], bo_ref[...]), 0.0)
    gl = lax.broadcasted_iota(jnp.int32, (1, NUM_GRAPHS), 1)
    onehot = (batch_ref[...] == gl).astype(jnp.float32)

    @pl.when(pl.program_id(0) == 0)
    def _():
        pool_ref[...] = jnp.zeros_like(pool_ref)

    pool_ref[...] += lax.dot_general(onehot, hn, (((0,), (0,)), ((), ())),
                                     preferred_element_type=jnp.float32)


_k3f = pl.pallas_call(
    _k3f_body,
    grid=(G,),
    in_specs=[_rows, _full((2, EMB)), _full((1, EMB)), _full((1, EMB)),
              pl.BlockSpec((B, 1), lambda i: (i, 0))],
    out_specs=_full((NUM_GRAPHS, EMB)),
    out_shape=jax.ShapeDtypeStruct((NUM_GRAPHS, EMB), jnp.float32),
)


def _head_body(pool_ref, wf1_ref, bf1_ref, wf2_ref, bf2_ref, out_ref):
    t = jnp.maximum(
        jnp.dot(pool_ref[...], wf1_ref[...], preferred_element_type=jnp.float32)
        + bf1_ref[...], 0.0)
    out_ref[...] = (jnp.dot(t, wf2_ref[...], preferred_element_type=jnp.float32)
                    + bf2_ref[...])


_head = pl.pallas_call(
    _head_body,
    out_shape=jax.ShapeDtypeStruct((NUM_GRAPHS, NUM_TASKS), jnp.float32),
)


def kernel(x, edge_index, edge_attr, batch, feat_table, W1, b1, g1, be1, W2, b2,
           eps, edge_tables, g_out, b_out, Wf1, bf1, Wf2, bf2):
    e = edge_index.shape[1]
    src = edge_index[0].astype(jnp.int32)
    dst = edge_index[1].astype(jnp.int32)
    idx = edge_attr.astype(jnp.int32) * N + src
    pad = jnp.arange(E_PAD - e, dtype=jnp.int32)
    idx_p = jnp.concatenate(
        [idx, pad % (NUM_EDGE_EMB * N)]).reshape(NW, K // CPR, CPR * CH)
    dst_p = jnp.concatenate(
        [dst, N + pad % (AGG_ROWS - N)]).reshape(NW, K // NB, 2, 2 * CH)
    zeros = jnp.zeros((AGG_ROWS, EMB), jnp.float32)
    x2 = x.astype(jnp.int32).reshape(N, 1)
    batch2 = batch.astype(jnp.int32).reshape(N, 1)

    h, r = _embed(x2, feat_table, edge_tables[0])
    for l in range(3):
        parts = _get_sc_scatter()(idx_p, dst_p, zeros,
                                  r.reshape(NUM_EDGE_EMB * N, EMB))
        eps_l = eps[l].reshape(1, 1)
        u, st1 = _k1(h, parts, eps_l, W1[l], b1[l].reshape(1, EMB))
        v, st2 = _k2(u, st1, g1[l].reshape(1, EMB), be1[l].reshape(1, EMB),
                     W2[l], b2[l].reshape(1, EMB))
        if l < 2:
            h, r = _k3(v, st2, g_out[l].reshape(1, EMB),
                       b_out[l].reshape(1, EMB), edge_tables[l + 1])
        else:
            pooled = _k3f(v, st2, g_out[l].reshape(1, EMB),
                          b_out[l].reshape(1, EMB), batch2)
    return _head(pooled, Wf1, bf1.reshape(1, 2 * EMB), Wf2,
                 bf2.reshape(1, NUM_TASKS))


# speedup vs baseline: 13.4151x; 1.0501x over previous
import functools

import jax
import jax.numpy as jnp
from jax import lax
from jax.experimental import pallas as pl
from jax.experimental.pallas import tpu as pltpu
from jax.experimental.pallas import tpu_sc as plsc

N = 10000
EMB = 128
IN_DIM = 128
NUM_EDGE_EMB = 4
NUM_GRAPHS = 128
NUM_TASKS = 10

NC, NS = 2, 16
NW = NC * NS
CH = 64
K = 160
NB = 4
CPR = 128 // CH
EPW = CH * K
E_PAD = EPW * NW
AGG_ROWS = 10112
ZR = AGG_ROWS // NS
RR = 640


def _sc_scatter_body(idx_hbm, dst_hbm, zeros_hbm, r_hbm, out_hbm,
                     idx_v, rows_v, dstb, sem_g, sem_s, sem_d, agg_sh):
    c = lax.axis_index("c")
    s = lax.axis_index("s")
    w = c * NS + s

    pltpu.sync_copy(zeros_hbm.at[pl.ds(s * ZR, ZR)], agg_sh.at[pl.ds(s * ZR, ZR)])
    pltpu.sync_copy(idx_hbm.at[w], idx_v)
    plsc.subcore_barrier()

    def fetch(j_row, part, b):
        pltpu.async_copy(
            r_hbm.at[idx_v.at[j_row, pl.ds(part * CH, CH)]], rows_v[b], sem_g[b])

    pltpu.async_copy(dst_hbm.at[w, 0], dstb[0], sem_d[0])
    for b in range(NB - 1):
        fetch(b // CPR, b % CPR, b)

    def cycle(i, parity):
        for b in range(NB):
            pltpu.make_async_copy(r_hbm.at[pl.ds(0, CH)], rows_v[b], sem_g[b]).wait()
            if b == 0:
                pltpu.make_async_copy(dst_hbm.at[0, 0], dstb[parity],
                                      sem_d[parity]).wait()
            pltpu.async_copy(
                rows_v[b],
                agg_sh.at[dstb[parity].at[b // 2, pl.ds((b % 2) * CH, CH)]],
                sem_s[b], add=True)
            bp = (b - 1) % NB
            jn = NB * i + b + (NB - 1)

            @pl.when(jn < K)
            def _():
                @pl.when(jn >= NB)
                def _():
                    pltpu.make_async_copy(r_hbm.at[pl.ds(0, CH)], rows_v[bp],
                                          sem_s[bp]).wait()

                fetch((NB // CPR) * i + (b + NB - 1) // CPR,
                      (b + NB - 1) % CPR, bp)

            if b == 0:
                @pl.when(i + 1 < K // NB)
                def _():
                    pltpu.async_copy(dst_hbm.at[w, i + 1], dstb[1 - parity],
                                     sem_d[1 - parity])

    def body2(t, carry):
        cycle(2 * t, 0)
        cycle(2 * t + 1, 1)
        return carry

    lax.fori_loop(0, K // NB // 2, body2, 0)
    for b in range(NB):
        pltpu.make_async_copy(r_hbm.at[pl.ds(0, CH)], rows_v[b], sem_s[b]).wait()
    plsc.subcore_barrier()

    @pl.when(s < NS - 1)
    def _():
        pltpu.sync_copy(agg_sh.at[pl.ds(s * RR, RR)],
                        out_hbm.at[c, pl.ds(s * RR, RR)])

    @pl.when(s == NS - 1)
    def _():
        pltpu.sync_copy(agg_sh.at[pl.ds((NS - 1) * RR, N - (NS - 1) * RR)],
                        out_hbm.at[c, pl.ds((NS - 1) * RR, N - (NS - 1) * RR)])


@functools.cache
def _get_sc_scatter():
    return pl.kernel(
        _sc_scatter_body,
        out_type=jax.ShapeDtypeStruct((NC, N, EMB), jnp.float32),
        mesh=plsc.VectorSubcoreMesh(core_axis_name="c", subcore_axis_name="s",
                                    num_cores=NC, num_subcores=NS),
        scratch_types=[
            pltpu.VMEM((K // CPR, CPR * CH), jnp.int32),
            [pltpu.VMEM((CH, EMB), jnp.float32) for _ in range(NB)],
            [pltpu.VMEM((2, 2 * CH), jnp.int32) for _ in range(2)],
            [pltpu.SemaphoreType.DMA for _ in range(NB)],
            [pltpu.SemaphoreType.DMA for _ in range(NB)],
            [pltpu.SemaphoreType.DMA for _ in range(2)],
            pltpu.VMEM_SHARED((AGG_ROWS, EMB), jnp.float32),
        ],
    )


B = 1000
G = N // B

_full = lambda shape: pl.BlockSpec(shape, lambda i: (0,) * len(shape))
_rows = pl.BlockSpec((B, EMB), lambda i: (i, 0))


def _embed_body(x_ref, ft_ref, tbl_ref, h_ref, r_ref):
    lanes = lax.broadcasted_iota(jnp.int32, (1, IN_DIM), 1)
    onehot = (x_ref[...] == lanes).astype(jnp.float32)
    h = jnp.dot(onehot, ft_ref[...], preferred_element_type=jnp.float32)
    h_ref[...] = h
    r_ref[...] = jnp.maximum(h[None] + tbl_ref[...][:, None, :], 0.0)


_embed = pl.pallas_call(
    _embed_body,
    grid=(G,),
    in_specs=[pl.BlockSpec((B, 1), lambda i: (i, 0)),
              _full((IN_DIM, EMB)), _full((NUM_EDGE_EMB, EMB))],
    out_specs=(_rows, pl.BlockSpec((NUM_EDGE_EMB, B, EMB), lambda i: (0, i, 0))),
    out_shape=(
        jax.ShapeDtypeStruct((N, EMB), jnp.float32),
        jax.ShapeDtypeStruct((NUM_EDGE_EMB, N, EMB), jnp.float32),
    ),
)


def _bn_from_stats(u, st, gamma, beta):
    mu = st[0:1] * (1.0 / N)
    var = st[1:2] * (1.0 / N) - mu * mu
    return (u - mu) / jnp.sqrt(var + 1e-5) * gamma + beta


_full2 = lambda shape: pl.BlockSpec(shape, lambda p, i: (0,) * len(shape))
_rows_p0 = pl.BlockSpec((B, EMB), lambda p, i: (jnp.where(p == 0, i, 0), 0))
_rows_p2 = pl.BlockSpec((B, EMB), lambda p, i: (jnp.where(p == 2, i, 0), 0))


def _layer_common(p_id, g, h_ref, p_ref, eps_ref, w1_ref, b1_ref, g1_ref,
                  be1_ref, w2_ref, b2_ref, go_ref, bo_ref, u_sc, v_sc, st1,
                  st2):
    @pl.when(p_id == 0)
    def _():
        z = (1.0 + eps_ref[0, 0]) * h_ref[...] + p_ref[0] + p_ref[1]
        u = jnp.dot(z, w1_ref[...], preferred_element_type=jnp.float32) \
            + b1_ref[...]
        u_sc[pl.ds(g * B, B)] = u

        @pl.when(g == 0)
        def _():
            st1[...] = jnp.zeros_like(st1)

        st1[0:1] += jnp.sum(u, axis=0, keepdims=True)
        st1[1:2] += jnp.sum(u * u, axis=0, keepdims=True)

    @pl.when(p_id == 1)
    def _():
        t = jnp.maximum(
            _bn_from_stats(u_sc[pl.ds(g * B, B)], st1[...], g1_ref[...],
                           be1_ref[...]), 0.0)
        v = jnp.dot(t, w2_ref[...], preferred_element_type=jnp.float32) \
            + b2_ref[...]
        v_sc[pl.ds(g * B, B)] = v

        @pl.when(g == 0)
        def _():
            st2[...] = jnp.zeros_like(st2)

        st2[0:1] += jnp.sum(v, axis=0, keepdims=True)
        st2[1:2] += jnp.sum(v * v, axis=0, keepdims=True)


def _layer_body(h_ref, p_ref, eps_ref, w1_ref, b1_ref, g1_ref, be1_ref,
                w2_ref, b2_ref, go_ref, bo_ref, tbl_ref, hn_ref, r_ref,
                u_sc, v_sc, st1, st2):
    p_id, g = pl.program_id(0), pl.program_id(1)
    _layer_common(p_id, g, h_ref, p_ref, eps_ref, w1_ref, b1_ref, g1_ref,
                  be1_ref, w2_ref, b2_ref, go_ref, bo_ref, u_sc, v_sc, st1,
                  st2)

    @pl.when(p_id == 2)
    def _():
        hn = jnp.maximum(
            _bn_from_stats(v_sc[pl.ds(g * B, B)], st2[...], go_ref[...],
                           bo_ref[...]), 0.0)
        hn_ref[...] = hn
        r_ref[...] = jnp.maximum(hn[None] + tbl_ref[...][:, None, :], 0.0)


_layer = pl.pallas_call(
    _layer_body,
    grid=(3, G),
    in_specs=[_rows_p0, pl.BlockSpec((NC, B, EMB),
                                     lambda p, i: (0, jnp.where(p == 0, i, 0), 0)),
              pl.BlockSpec(memory_space=pltpu.SMEM),
              _full2((EMB, EMB)), _full2((1, EMB)), _full2((1, EMB)),
              _full2((1, EMB)), _full2((EMB, EMB)), _full2((1, EMB)),
              _full2((1, EMB)), _full2((1, EMB)), _full2((NUM_EDGE_EMB, EMB))],
    out_specs=(_rows_p2,
               pl.BlockSpec((NUM_EDGE_EMB, B, EMB),
                            lambda p, i: (0, jnp.where(p == 2, i, 0), 0))),
    out_shape=(
        jax.ShapeDtypeStruct((N, EMB), jnp.float32),
        jax.ShapeDtypeStruct((NUM_EDGE_EMB, N, EMB), jnp.float32),
    ),
    scratch_shapes=[pltpu.VMEM((N, EMB), jnp.float32),
                    pltpu.VMEM((N, EMB), jnp.float32),
                    pltpu.VMEM((2, EMB), jnp.float32),
                    pltpu.VMEM((2, EMB), jnp.float32)],
)


def _layerf_body(h_ref, p_ref, eps_ref, w1_ref, b1_ref, g1_ref, be1_ref,
                 w2_ref, b2_ref, go_ref, bo_ref, batch_ref, pool_ref,
                 u_sc, v_sc, st1, st2):
    p_id, g = pl.program_id(0), pl.program_id(1)
    _layer_common(p_id, g, h_ref, p_ref, eps_ref, w1_ref, b1_ref, g1_ref,
                  be1_ref, w2_ref, b2_ref, go_ref, bo_ref, u_sc, v_sc, st1,
                  st2)

    @pl.when(p_id == 2)
    def _():
        hn = jnp.maximum(
            _bn_from_stats(v_sc[pl.ds(g * B, B)], st2[...], go_ref[...],
                           bo_ref[...]), 0.0)
        gl = lax.broadcasted_iota(jnp.int32, (1, NUM_GRAPHS), 1)
        onehot = (batch_ref[...] == gl).astype(jnp.float32)

        @pl.when(g == 0)
        def _():
            pool_ref[...] = jnp.zeros_like(pool_ref)

        pool_ref[...] += lax.dot_general(onehot, hn, (((0,), (0,)), ((), ())),
                                         preferred_element_type=jnp.float32)


_layerf = pl.pallas_call(
    _layerf_body,
    grid=(3, G),
    in_specs=[_rows_p0, pl.BlockSpec((NC, B, EMB),
                                     lambda p, i: (0, jnp.where(p == 0, i, 0), 0)),
              pl.BlockSpec(memory_space=pltpu.SMEM),
              _full2((EMB, EMB)), _full2((1, EMB)), _full2((1, EMB)),
              _full2((1, EMB)), _full2((EMB, EMB)), _full2((1, EMB)),
              _full2((1, EMB)), _full2((1, EMB)),
              pl.BlockSpec((B, 1), lambda p, i: (jnp.where(p == 2, i, 0), 0))],
    out_specs=_full2((NUM_GRAPHS, EMB)),
    out_shape=jax.ShapeDtypeStruct((NUM_GRAPHS, EMB), jnp.float32),
    scratch_shapes=[pltpu.VMEM((N, EMB), jnp.float32),
                    pltpu.VMEM((N, EMB), jnp.float32),
                    pltpu.VMEM((2, EMB), jnp.float32),
                    pltpu.VMEM((2, EMB), jnp.float32)],
)


def _head_body(pool_ref, wf1_ref, bf1_ref, wf2_ref, bf2_ref, out_ref):
    t = jnp.maximum(
        jnp.dot(pool_ref[...], wf1_ref[...], preferred_element_type=jnp.float32)
        + bf1_ref[...], 0.0)
    out_ref[...] = (jnp.dot(t, wf2_ref[...], preferred_element_type=jnp.float32)
                    + bf2_ref[...])


_head = pl.pallas_call(
    _head_body,
    out_shape=jax.ShapeDtypeStruct((NUM_GRAPHS, NUM_TASKS), jnp.float32),
)


def kernel(x, edge_index, edge_attr, batch, feat_table, W1, b1, g1, be1, W2, b2,
           eps, edge_tables, g_out, b_out, Wf1, bf1, Wf2, bf2):
    e = edge_index.shape[1]
    src = edge_index[0].astype(jnp.int32)
    dst = edge_index[1].astype(jnp.int32)
    idx = edge_attr.astype(jnp.int32) * N + src
    pad = jnp.arange(E_PAD - e, dtype=jnp.int32)
    idx_p = jnp.concatenate(
        [idx, pad % (NUM_EDGE_EMB * N)]).reshape(NW, K // CPR, CPR * CH)
    dst_p = jnp.concatenate(
        [dst, N + pad % (AGG_ROWS - N)]).reshape(NW, K // NB, 2, 2 * CH)
    zeros = jnp.zeros((AGG_ROWS, EMB), jnp.float32)
    x2 = x.astype(jnp.int32).reshape(N, 1)
    batch2 = batch.astype(jnp.int32).reshape(N, 1)

    h, r = _embed(x2, feat_table, edge_tables[0])
    for l in range(3):
        parts = _get_sc_scatter()(idx_p, dst_p, zeros,
                                  r.reshape(NUM_EDGE_EMB * N, EMB))
        eps_l = eps[l].reshape(1, 1)
        args = (h, parts, eps_l, W1[l], b1[l].reshape(1, EMB),
                g1[l].reshape(1, EMB), be1[l].reshape(1, EMB), W2[l],
                b2[l].reshape(1, EMB), g_out[l].reshape(1, EMB),
                b_out[l].reshape(1, EMB))
        if l < 2:
            h, r = _layer(*args, edge_tables[l + 1])
        else:
            pooled = _layerf(*args, batch2)
    return _head(pooled, Wf1, bf1.reshape(1, 2 * EMB), Wf2,
                 bf2.reshape(1, NUM_TASKS))


# speedup vs baseline: 13.4825x; 1.0050x over previous
import functools

import jax
import jax.numpy as jnp
from jax import lax
from jax.experimental import pallas as pl
from jax.experimental.pallas import tpu as pltpu
from jax.experimental.pallas import tpu_sc as plsc

N = 10000
EMB = 128
IN_DIM = 128
NUM_EDGE_EMB = 4
NUM_GRAPHS = 128
NUM_TASKS = 10

NC, NS = 2, 16
NW = NC * NS
CH = 64
K = 160
NB = 4
CPR = 128 // CH
EPW = CH * K
E_PAD = EPW * NW
AGG_ROWS = 10112
ZR = AGG_ROWS // NS
RR = 640


def _sc_scatter_body(idx_hbm, dst_hbm, zeros_hbm, r_hbm, out_hbm,
                     idx_v, rows_v, dstb, sem_g, sem_s, sem_d, agg_sh):
    c = lax.axis_index("c")
    s = lax.axis_index("s")
    w = c * NS + s

    pltpu.sync_copy(zeros_hbm.at[pl.ds(s * ZR, ZR)], agg_sh.at[pl.ds(s * ZR, ZR)])
    pltpu.sync_copy(idx_hbm.at[w], idx_v)
    plsc.subcore_barrier()

    def fetch(j_row, part, b):
        pltpu.async_copy(
            r_hbm.at[idx_v.at[j_row, pl.ds(part * CH, CH)]], rows_v[b], sem_g[b])

    pltpu.async_copy(dst_hbm.at[w, 0], dstb[0], sem_d[0])
    for b in range(NB - 1):
        fetch(b // CPR, b % CPR, b)

    def cycle(i, parity):
        for b in range(NB):
            pltpu.make_async_copy(r_hbm.at[pl.ds(0, CH)], rows_v[b], sem_g[b]).wait()
            if b == 0:
                pltpu.make_async_copy(dst_hbm.at[0, 0], dstb[parity],
                                      sem_d[parity]).wait()
            pltpu.async_copy(
                rows_v[b],
                agg_sh.at[dstb[parity].at[b // 2, pl.ds((b % 2) * CH, CH)]],
                sem_s[b], add=True)
            bp = (b - 1) % NB
            jn = NB * i + b + (NB - 1)

            @pl.when(jn < K)
            def _():
                @pl.when(jn >= NB)
                def _():
                    pltpu.make_async_copy(r_hbm.at[pl.ds(0, CH)], rows_v[bp],
                                          sem_s[bp]).wait()

                fetch((NB // CPR) * i + (b + NB - 1) // CPR,
                      (b + NB - 1) % CPR, bp)

            if b == 0:
                @pl.when(i + 1 < K // NB)
                def _():
                    pltpu.async_copy(dst_hbm.at[w, i + 1], dstb[1 - parity],
                                     sem_d[1 - parity])

    def body2(t, carry):
        cycle(2 * t, 0)
        cycle(2 * t + 1, 1)
        return carry

    lax.fori_loop(0, K // NB // 2, body2, 0)
    for b in range(NB):
        pltpu.make_async_copy(r_hbm.at[pl.ds(0, CH)], rows_v[b], sem_s[b]).wait()
    plsc.subcore_barrier()

    @pl.when(s < NS - 1)
    def _():
        pltpu.sync_copy(agg_sh.at[pl.ds(s * RR, RR)],
                        out_hbm.at[c, pl.ds(s * RR, RR)])

    @pl.when(s == NS - 1)
    def _():
        pltpu.sync_copy(agg_sh.at[pl.ds((NS - 1) * RR, N - (NS - 1) * RR)],
                        out_hbm.at[c, pl.ds((NS - 1) * RR, N - (NS - 1) * RR)])


@functools.cache
def _get_sc_scatter():
    return pl.kernel(
        _sc_scatter_body,
        out_type=jax.ShapeDtypeStruct((NC, N, EMB), jnp.float32),
        mesh=plsc.VectorSubcoreMesh(core_axis_name="c", subcore_axis_name="s",
                                    num_cores=NC, num_subcores=NS),
        scratch_types=[
            pltpu.VMEM((K // CPR, CPR * CH), jnp.int32),
            [pltpu.VMEM((CH, EMB), jnp.float32) for _ in range(NB)],
            [pltpu.VMEM((2, 2 * CH), jnp.int32) for _ in range(2)],
            [pltpu.SemaphoreType.DMA for _ in range(NB)],
            [pltpu.SemaphoreType.DMA for _ in range(NB)],
            [pltpu.SemaphoreType.DMA for _ in range(2)],
            pltpu.VMEM_SHARED((AGG_ROWS, EMB), jnp.float32),
        ],
    )


B = 1000
G = N // B

_full = lambda shape: pl.BlockSpec(shape, lambda i: (0,) * len(shape))
_rows = pl.BlockSpec((B, EMB), lambda i: (i, 0))


def _embed_body(x_ref, ft_ref, tbl_ref, h_ref, r_ref):
    lanes = lax.broadcasted_iota(jnp.int32, (1, IN_DIM), 1)
    onehot = (x_ref[...] == lanes).astype(jnp.float32)
    h = jnp.dot(onehot, ft_ref[...], preferred_element_type=jnp.float32)
    h_ref[...] = h
    r_ref[...] = jnp.maximum(h[None] + tbl_ref[...][:, None, :], 0.0)


_embed = pl.pallas_call(
    _embed_body,
    grid=(G,),
    in_specs=[pl.BlockSpec((B, 1), lambda i: (i, 0)),
              _full((IN_DIM, EMB)), _full((NUM_EDGE_EMB, EMB))],
    out_specs=(_rows, pl.BlockSpec((NUM_EDGE_EMB, B, EMB), lambda i: (0, i, 0))),
    out_shape=(
        jax.ShapeDtypeStruct((N, EMB), jnp.float32),
        jax.ShapeDtypeStruct((NUM_EDGE_EMB, N, EMB), jnp.float32),
    ),
)


def _bn_from_stats(u, st, gamma, beta):
    mu = st[0:1] * (1.0 / N)
    var = st[1:2] * (1.0 / N) - mu * mu
    return (u - mu) / jnp.sqrt(var + 1e-5) * gamma + beta


_full2 = lambda shape: pl.BlockSpec(shape, lambda p, i: (0,) * len(shape))
_rows_p0 = pl.BlockSpec((B, EMB), lambda p, i: (jnp.where(p == 0, i, 0), 0))
_rows_p2 = pl.BlockSpec((B, EMB), lambda p, i: (jnp.where(p == 2, i, 0), 0))


def _layer_common(p_id, g, h_ref, p_ref, eps_ref, w1_ref, b1_ref, g1_ref,
                  be1_ref, w2_ref, b2_ref, go_ref, bo_ref, u_sc, v_sc, st1,
                  st2):
    @pl.when(p_id == 0)
    def _():
        z = (1.0 + eps_ref[0, 0]) * h_ref[...] + p_ref[0] + p_ref[1]
        u = jnp.dot(z, w1_ref[...], preferred_element_type=jnp.float32) \
            + b1_ref[...]
        u_sc[pl.ds(g * B, B)] = u

        @pl.when(g == 0)
        def _():
            st1[...] = jnp.zeros_like(st1)

        st1[0:1] += jnp.sum(u, axis=0, keepdims=True)
        st1[1:2] += jnp.sum(u * u, axis=0, keepdims=True)

    @pl.when(p_id == 1)
    def _():
        t = jnp.maximum(
            _bn_from_stats(u_sc[pl.ds(g * B, B)], st1[...], g1_ref[...],
                           be1_ref[...]), 0.0)
        v = jnp.dot(t, w2_ref[...], preferred_element_type=jnp.float32) \
            + b2_ref[...]
        v_sc[pl.ds(g * B, B)] = v

        @pl.when(g == 0)
        def _():
            st2[...] = jnp.zeros_like(st2)

        st2[0:1] += jnp.sum(v, axis=0, keepdims=True)
        st2[1:2] += jnp.sum(v * v, axis=0, keepdims=True)


def _layer_body(h_ref, p_ref, eps_ref, w1_ref, b1_ref, g1_ref, be1_ref,
                w2_ref, b2_ref, go_ref, bo_ref, tbl_ref, hn_ref, r_ref,
                u_sc, v_sc, st1, st2):
    p_id, g = pl.program_id(0), pl.program_id(1)
    _layer_common(p_id, g, h_ref, p_ref, eps_ref, w1_ref, b1_ref, g1_ref,
                  be1_ref, w2_ref, b2_ref, go_ref, bo_ref, u_sc, v_sc, st1,
                  st2)

    @pl.when(p_id == 2)
    def _():
        hn = jnp.maximum(
            _bn_from_stats(v_sc[pl.ds(g * B, B)], st2[...], go_ref[...],
                           bo_ref[...]), 0.0)
        hn_ref[...] = hn
        r_ref[...] = jnp.maximum(hn[None] + tbl_ref[...][:, None, :], 0.0)


_layer = pl.pallas_call(
    _layer_body,
    grid=(3, G),
    in_specs=[_rows_p0, pl.BlockSpec((NC, B, EMB),
                                     lambda p, i: (0, jnp.where(p == 0, i, 0), 0)),
              pl.BlockSpec(memory_space=pltpu.SMEM),
              _full2((EMB, EMB)), _full2((1, EMB)), _full2((1, EMB)),
              _full2((1, EMB)), _full2((EMB, EMB)), _full2((1, EMB)),
              _full2((1, EMB)), _full2((1, EMB)), _full2((NUM_EDGE_EMB, EMB))],
    out_specs=(_rows_p2,
               pl.BlockSpec((NUM_EDGE_EMB, B, EMB),
                            lambda p, i: (0, jnp.where(p == 2, i, 0), 0))),
    out_shape=(
        jax.ShapeDtypeStruct((N, EMB), jnp.float32),
        jax.ShapeDtypeStruct((NUM_EDGE_EMB, N, EMB), jnp.float32),
    ),
    scratch_shapes=[pltpu.VMEM((N, EMB), jnp.float32),
                    pltpu.VMEM((N, EMB), jnp.float32),
                    pltpu.VMEM((2, EMB), jnp.float32),
                    pltpu.VMEM((2, EMB), jnp.float32)],
)


def _layerf_body(h_ref, p_ref, eps_ref, w1_ref, b1_ref, g1_ref, be1_ref,
                 w2_ref, b2_ref, go_ref, bo_ref, batch_ref, wf1_ref, bf1_ref,
                 wf2_ref, bf2_ref, out_ref, u_sc, v_sc, st1, st2, pool_sc):
    p_id, g = pl.program_id(0), pl.program_id(1)
    _layer_common(p_id, g, h_ref, p_ref, eps_ref, w1_ref, b1_ref, g1_ref,
                  be1_ref, w2_ref, b2_ref, go_ref, bo_ref, u_sc, v_sc, st1,
                  st2)

    @pl.when(p_id == 2)
    def _():
        hn = jnp.maximum(
            _bn_from_stats(v_sc[pl.ds(g * B, B)], st2[...], go_ref[...],
                           bo_ref[...]), 0.0)
        gl = lax.broadcasted_iota(jnp.int32, (1, NUM_GRAPHS), 1)
        onehot = (batch_ref[...] == gl).astype(jnp.float32)

        @pl.when(g == 0)
        def _():
            pool_sc[...] = jnp.zeros_like(pool_sc)

        pool_sc[...] += lax.dot_general(onehot, hn, (((0,), (0,)), ((), ())),
                                        preferred_element_type=jnp.float32)

        @pl.when(g == G - 1)
        def _():
            t = jnp.maximum(
                jnp.dot(pool_sc[...], wf1_ref[...],
                        preferred_element_type=jnp.float32) + bf1_ref[...],
                0.0)
            out_ref[...] = (jnp.dot(t, wf2_ref[...],
                                    preferred_element_type=jnp.float32)
                            + bf2_ref[...])


_layerf = pl.pallas_call(
    _layerf_body,
    grid=(3, G),
    in_specs=[_rows_p0, pl.BlockSpec((NC, B, EMB),
                                     lambda p, i: (0, jnp.where(p == 0, i, 0), 0)),
              pl.BlockSpec(memory_space=pltpu.SMEM),
              _full2((EMB, EMB)), _full2((1, EMB)), _full2((1, EMB)),
              _full2((1, EMB)), _full2((EMB, EMB)), _full2((1, EMB)),
              _full2((1, EMB)), _full2((1, EMB)),
              pl.BlockSpec((B, 1), lambda p, i: (jnp.where(p == 2, i, 0), 0)),
              _full2((EMB, 2 * EMB)), _full2((1, 2 * EMB)),
              _full2((2 * EMB, NUM_TASKS)), _full2((1, NUM_TASKS))],
    out_specs=_full2((NUM_GRAPHS, NUM_TASKS)),
    out_shape=jax.ShapeDtypeStruct((NUM_GRAPHS, NUM_TASKS), jnp.float32),
    scratch_shapes=[pltpu.VMEM((N, EMB), jnp.float32),
                    pltpu.VMEM((N, EMB), jnp.float32),
                    pltpu.VMEM((2, EMB), jnp.float32),
                    pltpu.VMEM((2, EMB), jnp.float32),
                    pltpu.VMEM((NUM_GRAPHS, EMB), jnp.float32)],
)


def kernel(x, edge_index, edge_attr, batch, feat_table, W1, b1, g1, be1, W2, b2,
           eps, edge_tables, g_out, b_out, Wf1, bf1, Wf2, bf2):
    e = edge_index.shape[1]
    src = edge_index[0].astype(jnp.int32)
    dst = edge_index[1].astype(jnp.int32)
    idx = edge_attr.astype(jnp.int32) * N + src
    pad = jnp.arange(E_PAD - e, dtype=jnp.int32)
    idx_p = jnp.concatenate(
        [idx, pad % (NUM_EDGE_EMB * N)]).reshape(NW, K // CPR, CPR * CH)
    dst_p = jnp.concatenate(
        [dst, N + pad % (AGG_ROWS - N)]).reshape(NW, K // NB, 2, 2 * CH)
    zeros = jnp.zeros((AGG_ROWS, EMB), jnp.float32)
    x2 = x.astype(jnp.int32).reshape(N, 1)
    batch2 = batch.astype(jnp.int32).reshape(N, 1)

    h, r = _embed(x2, feat_table, edge_tables[0])
    for l in range(3):
        parts = _get_sc_scatter()(idx_p, dst_p, zeros,
                                  r.reshape(NUM_EDGE_EMB * N, EMB))
        eps_l = eps[l].reshape(1, 1)
        args = (h, parts, eps_l, W1[l], b1[l].reshape(1, EMB),
                g1[l].reshape(1, EMB), be1[l].reshape(1, EMB), W2[l],
                b2[l].reshape(1, EMB), g_out[l].reshape(1, EMB),
                b_out[l].reshape(1, EMB))
        if l < 2:
            h, r = _layer(*args, edge_tables[l + 1])
        else:
            out = _layerf(*args, batch2, Wf1, bf1.reshape(1, 2 * EMB), Wf2,
                          bf2.reshape(1, NUM_TASKS))
    return out
